# Initial kernel scaffold; baseline (speedup 1.0000x reference)
import jax
import jax.numpy as jnp
from jax.experimental import pallas as pl


def kernel(x, edge_index, edge_attr, lin1_w, lin1_b, conv_w, lp1_w, lp1_b, lp2_w, lp2_b):
    raise NotImplementedError("write your pallas kernel here")


# speedup vs baseline: 4.0495x; 4.0495x over previous
import functools

import jax
import jax.numpy as jnp
import numpy as np
from jax import lax
from jax.experimental import pallas as pl
from jax.experimental.pallas import tpu as pltpu
from jax.experimental.pallas import tpu_sc as plsc

ALPHA = 0.1
THETA = 1.5

NC = 2
NS = 16
LN = 16
CH = 16
QW = 128
PRT = 4
PBITS = 14


def _pad_to(a, n, fill):
    if a.shape[0] == n:
        return a
    return jnp.concatenate(
        [a, jnp.full((n - a.shape[0],), fill, dtype=a.dtype)])


def kernel(x, edge_index, edge_attr, lin1_w, lin1_b, conv_w,
           lp1_w, lp1_b, lp2_w, lp2_b):
    N, D_IN = x.shape
    H = lin1_w.shape[1]
    L = conv_w.shape[0]
    C = lp2_w.shape[1]
    E = edge_attr.shape[0]
    assert H == 4 * QW and N % NS == 0 and N <= (1 << PBITS)

    loop = jnp.arange(N, dtype=edge_index.dtype)
    src2 = jnp.concatenate([edge_index[0], loop])
    dst2 = jnp.concatenate([edge_index[1], loop])
    w2 = jnp.concatenate([edge_attr, jnp.ones((N,), dtype=x.dtype)])

    E2 = E + N
    nch = -(-E2 // (NS * CH))
    nch = -(-nch // (2 * PRT)) * (2 * PRT)
    NCH = nch
    PCH = NCH // PRT
    PE2 = NS * NCH * CH
    TPT1 = PE2 // NS
    TPT2 = PE2 // (NS * NC)

    srcp = _pad_to(src2, PE2, 0)
    dstp = _pad_to(dst2, PE2, 0)
    wp = _pad_to(w2, PE2, 0.0)
    encp = jnp.bitwise_or(srcp, jnp.left_shift(dstp, PBITS))

    WT = -(-N // NS)
    WT = -(-WT // LN) * LN
    NP = NS * WT

    f32 = jnp.float32
    mesh = plsc.VectorSubcoreMesh(core_axis_name="c", subcore_axis_name="s",
                                  num_cores=NC, num_subcores=NS)
    sc_params = pltpu.CompilerParams(needs_layout_passes=False)

    @functools.partial(
        pl.kernel,
        out_type=jax.ShapeDtypeStruct((NS, NP), f32),
        mesh=mesh,
        compiler_params=sc_params,
        scratch_types=[
            pltpu.VMEM((NP,), f32),
            pltpu.VMEM((TPT1,), jnp.int32),
            pltpu.VMEM((TPT1,), f32),
        ],
    )
    def _deg_kernel(col_hbm, w_hbm, degp_hbm, deg_v, colb, wb):
        c = lax.axis_index("c")
        s = lax.axis_index("s")
        zero16 = jnp.zeros((LN,), f32)

        @pl.when(c == 0)
        def _():
            @pl.loop(0, NP // LN)
            def _(r):
                deg_v[pl.ds(r * LN, LN)] = zero16

            pltpu.sync_copy(col_hbm.at[pl.ds(s * TPT1, TPT1)], colb)
            pltpu.sync_copy(w_hbm.at[pl.ds(s * TPT1, TPT1)], wb)

            @pl.loop(0, TPT1 // LN)
            def _(g):
                idx = colb[pl.ds(g * LN, LN)]
                wv = wb[pl.ds(g * LN, LN)]
                plsc.addupdate_scatter(deg_v, [idx], wv)

            pltpu.sync_copy(deg_v, degp_hbm.at[s])

    @functools.partial(
        pl.kernel,
        out_type=jax.ShapeDtypeStruct((PE2,), f32),
        mesh=mesh,
        compiler_params=sc_params,
        scratch_types=[
            pltpu.VMEM((NP,), f32),
            pltpu.VMEM((NP,), f32),
            pltpu.VMEM((NP,), f32),
            pltpu.VMEM((TPT2,), jnp.int32),
            pltpu.VMEM((TPT2,), f32),
            pltpu.VMEM((TPT2,), f32),
        ],
    )
    def _norm_kernel(degp_hbm, enc_hbm, w_hbm, nrm_hbm,
                     deg_v, dinv_v, tmp_v, encb, wb, nrmb):
        c = lax.axis_index("c")
        s = lax.axis_index("s")
        wid = c * NS + s
        pltpu.sync_copy(degp_hbm.at[0], deg_v)
        for r in range(1, NS):
            pltpu.sync_copy(degp_hbm.at[r], tmp_v)

            @pl.loop(0, NP // LN)
            def _(g):
                sl = pl.ds(g * LN, LN)
                deg_v[sl] = deg_v[sl] + tmp_v[sl]

        @pl.loop(0, NP // LN)
        def _(g):
            d = deg_v[pl.ds(g * LN, LN)]
            dm = jnp.maximum(d, 1e-12)
            bits = plsc.bitcast(dm, jnp.int32)
            i0 = jnp.int32(0x5F3759DF) - lax.shift_right_logical(bits, 1)
            xx = plsc.bitcast(i0, f32)
            for _ in range(3):
                xx = xx * (1.5 - 0.5 * dm * xx * xx)
            dinv_v[pl.ds(g * LN, LN)] = jnp.where(d > 0, xx, 0.0)

        base = wid * TPT2
        pltpu.sync_copy(enc_hbm.at[pl.ds(base, TPT2)], encb)
        pltpu.sync_copy(w_hbm.at[pl.ds(base, TPT2)], wb)

        @pl.loop(0, TPT2 // LN)
        def _(g):
            enc = encb[pl.ds(g * LN, LN)]
            si = jnp.bitwise_and(enc, (1 << PBITS) - 1)
            di = lax.shift_right_logical(enc, PBITS)
            wv = wb[pl.ds(g * LN, LN)]
            da = plsc.load_gather(dinv_v, [si])
            db = plsc.load_gather(dinv_v, [di])
            nrmb[pl.ds(g * LN, LN)] = da * wv * db

        pltpu.sync_copy(nrmb, nrm_hbm.at[pl.ds(base, TPT2)])

    RPT = NP // NS
    assert RPT % CH == 0

    @functools.partial(
        pl.kernel,
        out_type=jax.ShapeDtypeStruct((4, NP, QW), f32),
        mesh=mesh,
        compiler_params=sc_params,
        scratch_types=[
            pltpu.VMEM((PCH, CH), jnp.int32),
            pltpu.VMEM((PCH, CH), f32),
            pltpu.VMEM((CH, QW), f32),
            pltpu.VMEM((CH, QW), f32),
            pltpu.VMEM_SHARED((NP, QW), f32),
            pltpu.SemaphoreType.DMA,
            pltpu.SemaphoreType.DMA,
        ],
    )
    def _agg_kernel(x4_hbm, enc_hbm, nrm_hbm, z4_hbm,
                    enc_v, nrm_v, rowsA, rowsB, zacc, gsA, gsB):
        c = lax.axis_index("c")
        s = lax.axis_index("s")
        zero16 = jnp.zeros((LN,), f32)
        mask = (1 << PBITS) - 1

        def srcvec(i):
            return jnp.bitwise_and(enc_v[i, pl.ds(0, CH)], mask)

        def start_gather(q, i, buf, sem):
            pltpu.async_copy(x4_hbm.at[q].at[srcvec(i)], buf, sem)

        def process(q, i, buf, sem):
            pltpu.make_async_copy(x4_hbm.at[q].at[srcvec(i)], buf,
                                  sem).wait()
            nv = nrm_v[i, pl.ds(0, CH)]
            for j in range(CH):
                w = nv[j]
                for k in range(QW // LN):
                    sl = pl.ds(k * LN, LN)
                    buf[j, sl] = buf[j, sl] * w
            dstv = lax.shift_right_logical(enc_v[i, pl.ds(0, CH)], PBITS)
            pltpu.sync_copy(buf, zacc.at[dstv], add=True)

            @pl.when(i + 2 < PCH)
            def _():
                start_gather(q, i + 2, buf, sem)

        def do_quarter(q):
            @pl.loop(0, CH)
            def _(r):
                for k in range(QW // LN):
                    rowsA[r, pl.ds(k * LN, LN)] = zero16

            for p in range(RPT // CH):
                pltpu.sync_copy(rowsA,
                                zacc.at[pl.ds(s * RPT + p * CH, CH)])
            plsc.subcore_barrier()
            for pt in range(PRT):
                pltpu.sync_copy(enc_hbm.at[s, pl.ds(pt * PCH, PCH)], enc_v)
                pltpu.sync_copy(nrm_hbm.at[s, pl.ds(pt * PCH, PCH)], nrm_v)
                start_gather(q, 0, rowsA, gsA)
                start_gather(q, 1, rowsB, gsB)

                @pl.loop(0, PCH, step=2)
                def _(i):
                    process(q, i, rowsA, gsA)
                    process(q, i + 1, rowsB, gsB)

            plsc.subcore_barrier()
            for p in range(RPT // CH):
                sl = pl.ds(s * RPT + p * CH, CH)
                pltpu.sync_copy(zacc.at[sl], rowsA)
                pltpu.sync_copy(rowsA, z4_hbm.at[q, sl])
            plsc.subcore_barrier()

        for q in range(4):
            @pl.when(c == q // 2)
            def _(q=q):
                do_quarter(q)

    BR = 400
    GRID = N // BR

    def _quarters(xc):
        return jnp.stack([xc[:, q * QW:(q + 1) * QW] for q in range(4)],
                         axis=0)

    def _lin1_body(x_ref, w_ref, b_ref, h_ref, q4_ref):
        h = jnp.dot(x_ref[...], w_ref[...],
                    preferred_element_type=f32) + b_ref[...]
        h_ref[...] = h
        q4_ref[...] = _quarters(h)

    lin1_call = pl.pallas_call(
        _lin1_body,
        grid=(GRID,),
        in_specs=[
            pl.BlockSpec((BR, D_IN), lambda i: (i, 0)),
            pl.BlockSpec((D_IN, H), lambda i: (0, 0)),
            pl.BlockSpec((1, H), lambda i: (0, 0)),
        ],
        out_specs=[
            pl.BlockSpec((BR, H), lambda i: (i, 0)),
            pl.BlockSpec((4, BR, QW), lambda i: (0, i, 0)),
        ],
        out_shape=[
            jax.ShapeDtypeStruct((N, H), f32),
            jax.ShapeDtypeStruct((4, N, QW), f32),
        ],
    )

    def _layer_body(beta, last, z4_ref, x1_ref, w_ref, *outs):
        agg = jnp.concatenate([z4_ref[q] for q in range(4)], axis=1)
        out = agg * (1.0 - ALPHA) + ALPHA * x1_ref[...]
        t = (1.0 - beta) * out + beta * jnp.dot(
            out, w_ref[...], preferred_element_type=f32)
        xc = jnp.maximum(t, 0.0)
        outs[0][...] = _quarters(xc)
        if last:
            outs[1][...] = xc

    def _layer_call(beta, last):
        out_specs = [pl.BlockSpec((4, BR, QW), lambda i: (0, i, 0))]
        out_shape = [jax.ShapeDtypeStruct((4, N, QW), f32)]
        if last:
            out_specs.append(pl.BlockSpec((BR, H), lambda i: (i, 0)))
            out_shape.append(jax.ShapeDtypeStruct((N, H), f32))
        return pl.pallas_call(
            functools.partial(_layer_body, beta, last),
            grid=(GRID,),
            in_specs=[
                pl.BlockSpec((4, BR, QW), lambda i: (0, i, 0)),
                pl.BlockSpec((BR, H), lambda i: (i, 0)),
                pl.BlockSpec((H, H), lambda i: (0, 0)),
            ],
            out_specs=out_specs,
            out_shape=out_shape,
        )

    def _head_body(xc_ref, w1_ref, b1_ref, w2_ref, b2_ref, o_ref):
        p = jnp.maximum(
            jnp.dot(xc_ref[...], w1_ref[...],
                    preferred_element_type=f32) + b1_ref[...], 0.0)
        lg = jnp.dot(p, w2_ref[...],
                     preferred_element_type=f32) + b2_ref[...]
        m = jnp.max(lg, axis=1, keepdims=True)
        e = jnp.exp(lg - m)
        o_ref[...] = lg - m - jnp.log(jnp.sum(e, axis=1, keepdims=True))

    head_call = pl.pallas_call(
        _head_body,
        grid=(GRID,),
        in_specs=[
            pl.BlockSpec((BR, H), lambda i: (i, 0)),
            pl.BlockSpec((H, H), lambda i: (0, 0)),
            pl.BlockSpec((1, H), lambda i: (0, 0)),
            pl.BlockSpec((H, C), lambda i: (0, 0)),
            pl.BlockSpec((1, C), lambda i: (0, 0)),
        ],
        out_specs=pl.BlockSpec((BR, C), lambda i: (i, 0)),
        out_shape=jax.ShapeDtypeStruct((N, C), f32),
    )

    deg = _deg_kernel(dstp, wp)
    nrm = _norm_kernel(deg, encp, wp)

    enc3 = encp.reshape(NS, NCH, CH)
    nrm3 = nrm.reshape(NS, NCH, CH)

    x1, xq4 = lin1_call(x, lin1_w, lin1_b.reshape(1, H))
    for l in range(L):
        z4 = _agg_kernel(xq4, enc3, nrm3)
        beta = float(np.log(THETA / (l + 1) + 1.0))
        res = _layer_call(beta, l == L - 1)(z4, x1, conv_w[l])
        xq4 = res[0]
        if l == L - 1:
            xc_full = res[1]

    return head_call(xc_full, lp1_w, lp1_b.reshape(1, H),
                     lp2_w, lp2_b.reshape(1, C))
